# baseline (speedup 1.0000x reference)
import jax
import jax.numpy as jnp
from jax.experimental import pallas as pl
from jax.experimental.pallas import tpu as pltpu

_NBUF = 4
_TI = 4


def _body(tok_ref, en_ref, w_ref, eb_ref, pe_ref, out_ref,
          in_buf, out_buf, e_buf, tin, tout,
          in_sems, out_sems, e_sem, t_sems):
    n_in = tok_ref.shape[0]
    batch = tok_ref.shape[2]
    nc = n_in // _TI
    rem = n_in - nc * _TI

    def in_copy(ci, slot):
        return pltpu.make_async_copy(
            tok_ref.at[pl.ds(ci * _TI, _TI)], in_buf.at[slot],
            in_sems.at[slot])

    def out_copy(ci, slot):
        return pltpu.make_async_copy(
            out_buf.at[slot], out_ref.at[pl.ds(ci * _TI + 1, _TI)],
            out_sems.at[slot])

    for s in range(_NBUF):
        in_copy(s, s).start()
    t_in = pltpu.make_async_copy(
        tok_ref.at[pl.ds(nc * _TI, rem)], tin, t_sems.at[0])
    t_in.start()
    qtr = batch // 4
    for j in range(4):
        sl = pl.ds(j * qtr, qtr)
        e_buf[:, sl] = jax.lax.dot_general(
            w_ref[:], en_ref[:, sl], (((0,), (0,)), ((), ())),
            preferred_element_type=jnp.float32) + eb_ref[:]
    e_cp = pltpu.make_async_copy(e_buf, out_ref.at[0], e_sem)
    e_cp.start()

    def chunk(ci, slot, first_round):
        in_copy(ci, slot).wait()
        if not first_round:
            out_copy(ci - _NBUF, slot).wait()
        out_buf[slot] = in_buf[slot] + pe_ref[pl.ds(ci * _TI, _TI)]
        out_copy(ci, slot).start()

        @pl.when(ci + _NBUF < nc)
        def _():
            in_copy(ci + _NBUF, slot).start()

    for s in range(_NBUF):
        chunk(s, s, True)

    n_rounds = (nc - _NBUF) // _NBUF

    def round_body(r, carry):
        base = (r + 1) * _NBUF
        for s in range(_NBUF):
            chunk(base + s, s, False)
        return carry

    jax.lax.fori_loop(0, n_rounds, round_body, 0)

    for ci in range((n_rounds + 1) * _NBUF, nc):
        chunk(ci, ci % _NBUF, False)

    t_in.wait()
    tout[:] = tin[:] + pe_ref[pl.ds(nc * _TI, rem)]
    t_out = pltpu.make_async_copy(
        tout, out_ref.at[pl.ds(nc * _TI + 1, rem)], t_sems.at[1])
    t_out.start()

    for ci in range(nc - _NBUF, nc):
        out_copy(ci, ci % _NBUF).wait()
    t_out.wait()
    e_cp.wait()


def kernel(tokens, energies, W, b, emb):
    batch, n_in, tsz = tokens.shape
    n_tok = emb.shape[0]
    tokens_t = tokens.transpose(1, 2, 0)
    energies_t = energies.T
    pe = emb[1:].reshape(n_in, tsz, 1)
    eb = (b + emb[0]).reshape(tsz, 1)
    rem = n_in - (n_in // _TI) * _TI

    resident = pl.BlockSpec(memory_space=pltpu.MemorySpace.VMEM)
    hbm = pl.BlockSpec(memory_space=pl.ANY)
    out_t = pl.pallas_call(
        _body,
        in_specs=[hbm, resident, resident, resident, resident],
        out_specs=hbm,
        out_shape=jax.ShapeDtypeStruct((n_tok, tsz, batch), jnp.float32),
        scratch_shapes=[
            pltpu.VMEM((_NBUF, _TI, tsz, batch), jnp.float32),
            pltpu.VMEM((_NBUF, _TI, tsz, batch), jnp.float32),
            pltpu.VMEM((tsz, batch), jnp.float32),
            pltpu.VMEM((rem, tsz, batch), jnp.float32),
            pltpu.VMEM((rem, tsz, batch), jnp.float32),
            pltpu.SemaphoreType.DMA((_NBUF,)),
            pltpu.SemaphoreType.DMA((_NBUF,)),
            pltpu.SemaphoreType.DMA,
            pltpu.SemaphoreType.DMA((2,)),
        ],
    )(tokens_t, energies_t, W, eb, pe)
    return out_t.transpose(2, 0, 1)


# speedup vs baseline: 1.0002x; 1.0002x over previous
import jax
import jax.numpy as jnp
from jax.experimental import pallas as pl
from jax.experimental.pallas import tpu as pltpu

_NBUF = 4
_TI = 4


def _body(tok_ref, en_ref, w_ref, eb_ref, pe_ref, out_ref,
          in_buf, out_buf, e_buf, tin, tout,
          in_sems, out_sems, e_sem, t_sems):
    n_in = tok_ref.shape[0]
    batch = tok_ref.shape[2]
    nc = n_in // _TI
    rem = n_in - nc * _TI

    half = batch // 2

    class _Pair:
        def __init__(self, a, b):
            self._a, self._b = a, b

        def start(self):
            self._a.start()
            self._b.start()

        def wait(self):
            self._a.wait()
            self._b.wait()

    def in_copy(ci, slot):
        src = tok_ref.at[pl.ds(ci * _TI, _TI)]
        return _Pair(
            pltpu.make_async_copy(src.at[:, :, pl.ds(0, half)],
                                  in_buf.at[slot, :, :, pl.ds(0, half)],
                                  in_sems.at[slot, 0]),
            pltpu.make_async_copy(src.at[:, :, pl.ds(half, half)],
                                  in_buf.at[slot, :, :, pl.ds(half, half)],
                                  in_sems.at[slot, 1]))

    def out_copy(ci, slot):
        dst = out_ref.at[pl.ds(ci * _TI + 1, _TI)]
        return _Pair(
            pltpu.make_async_copy(out_buf.at[slot, :, :, pl.ds(0, half)],
                                  dst.at[:, :, pl.ds(0, half)],
                                  out_sems.at[slot, 0]),
            pltpu.make_async_copy(out_buf.at[slot, :, :, pl.ds(half, half)],
                                  dst.at[:, :, pl.ds(half, half)],
                                  out_sems.at[slot, 1]))

    for s in range(_NBUF):
        in_copy(s, s).start()
    t_in = pltpu.make_async_copy(
        tok_ref.at[pl.ds(nc * _TI, rem)], tin, t_sems.at[0])
    t_in.start()
    qtr = batch // 4
    for j in range(4):
        sl = pl.ds(j * qtr, qtr)
        e_buf[:, sl] = jax.lax.dot_general(
            w_ref[:], en_ref[:, sl], (((0,), (0,)), ((), ())),
            preferred_element_type=jnp.float32) + eb_ref[:]
    e_cp = pltpu.make_async_copy(e_buf, out_ref.at[0], e_sem)
    e_cp.start()

    def chunk(ci, slot, first_round):
        in_copy(ci, slot).wait()
        if not first_round:
            out_copy(ci - _NBUF, slot).wait()
        out_buf[slot] = in_buf[slot] + pe_ref[pl.ds(ci * _TI, _TI)]
        out_copy(ci, slot).start()

        @pl.when(ci + _NBUF < nc)
        def _():
            in_copy(ci + _NBUF, slot).start()

    for s in range(_NBUF):
        chunk(s, s, True)

    n_rounds = (nc - _NBUF) // _NBUF

    def round_body(r, carry):
        base = (r + 1) * _NBUF
        for s in range(_NBUF):
            chunk(base + s, s, False)
        return carry

    jax.lax.fori_loop(0, n_rounds, round_body, 0)

    for ci in range((n_rounds + 1) * _NBUF, nc):
        chunk(ci, ci % _NBUF, False)

    t_in.wait()
    tout[:] = tin[:] + pe_ref[pl.ds(nc * _TI, rem)]
    t_out = pltpu.make_async_copy(
        tout, out_ref.at[pl.ds(nc * _TI + 1, rem)], t_sems.at[1])
    t_out.start()

    for ci in range(nc - _NBUF, nc):
        out_copy(ci, ci % _NBUF).wait()
    t_out.wait()
    e_cp.wait()


def kernel(tokens, energies, W, b, emb):
    batch, n_in, tsz = tokens.shape
    n_tok = emb.shape[0]
    tokens_t = tokens.transpose(1, 2, 0)
    energies_t = energies.T
    pe = emb[1:].reshape(n_in, tsz, 1)
    eb = (b + emb[0]).reshape(tsz, 1)
    rem = n_in - (n_in // _TI) * _TI

    resident = pl.BlockSpec(memory_space=pltpu.MemorySpace.VMEM)
    hbm = pl.BlockSpec(memory_space=pl.ANY)
    out_t = pl.pallas_call(
        _body,
        in_specs=[hbm, resident, resident, resident, resident],
        out_specs=hbm,
        out_shape=jax.ShapeDtypeStruct((n_tok, tsz, batch), jnp.float32),
        scratch_shapes=[
            pltpu.VMEM((_NBUF, _TI, tsz, batch), jnp.float32),
            pltpu.VMEM((_NBUF, _TI, tsz, batch), jnp.float32),
            pltpu.VMEM((tsz, batch), jnp.float32),
            pltpu.VMEM((rem, tsz, batch), jnp.float32),
            pltpu.VMEM((rem, tsz, batch), jnp.float32),
            pltpu.SemaphoreType.DMA((_NBUF, 2)),
            pltpu.SemaphoreType.DMA((_NBUF, 2)),
            pltpu.SemaphoreType.DMA,
            pltpu.SemaphoreType.DMA((2,)),
        ],
    )(tokens_t, energies_t, W, eb, pe)
    return out_t.transpose(2, 0, 1)


# speedup vs baseline: 1.0073x; 1.0071x over previous
import jax
import jax.numpy as jnp
from jax.experimental import pallas as pl
from jax.experimental.pallas import tpu as pltpu

_BL = 256


def _body(tok_ref, en_ref, w_ref, eb_ref, pe_ref, out_ref):
    j = pl.program_id(0)
    e = jax.lax.dot_general(
        w_ref[:], en_ref[:, pl.ds(j * _BL, _BL)], (((0,), (0,)), ((), ())),
        preferred_element_type=jnp.float32)
    out_ref[0, :, :] = e + eb_ref[:]
    out_ref[1:, :, :] = tok_ref[:] + pe_ref[:]


def kernel(tokens, energies, W, b, emb):
    batch, n_in, tsz = tokens.shape
    n_tok = emb.shape[0]
    tokens_t = tokens.transpose(1, 2, 0)
    energies_t = energies.T
    pe = emb[1:].reshape(n_in, tsz, 1)
    eb = (b + emb[0]).reshape(tsz, 1)

    grid = (batch // _BL,)
    resident = pl.BlockSpec(memory_space=pltpu.MemorySpace.VMEM)
    out_t = pl.pallas_call(
        _body,
        grid=grid,
        in_specs=[
            pl.BlockSpec((n_in, tsz, _BL), lambda j: (0, 0, j)),
            resident,
            resident,
            resident,
            resident,
        ],
        out_specs=pl.BlockSpec((n_tok, tsz, _BL), lambda j: (0, 0, j)),
        out_shape=jax.ShapeDtypeStruct((n_tok, tsz, batch), jnp.float32),
    )(tokens_t, energies_t, W, eb, pe)
    return out_t.transpose(2, 0, 1)
